# baseline (speedup 1.0000x reference)
import functools

import jax
import jax.numpy as jnp
from jax import lax
from jax.experimental import pallas as pl
from jax.experimental.pallas import tpu as pltpu
from jax.experimental.pallas import tpu_sc as plsc

N = 10000
D = 256
E = 160000
HALF = 128
NC = 2
NS = 16
NW = NC * NS

ND = 10240
NDS = ND // NS
EPW = E // NW
CH = 128
DFULL = EPW // CH
DTAIL = EPW - DFULL * CH

EPS = E // NS
NFULL = EPS // CH
TAIL = EPS - NFULL * CH
RPT = 624
RREM = N - NS * RPT

RB = 1000
NB = N // RB

@functools.cache
def _mesh():
    return plsc.VectorSubcoreMesh(core_axis_name="c", subcore_axis_name="s",
                                  num_cores=NC, num_subcores=NS)


_DCH = 2 * DFULL
_DSL = 2 * ND // NS


def _deg_body(dst2, out, deg_sh, idx0, idx1, idxtA, idxtB, ones_v, zero_v,
              semi0, semi1):
    c = lax.axis_index("c")
    s = lax.axis_index("s")
    w = s * NC + c
    wE = w * EPW

    def fill_ones(i, _):
        ones_v[pl.ds(i * 16, 16)] = jnp.full((16,), 1.0, jnp.float32)
        return 0

    lax.fori_loop(0, CH // 16, fill_ones, 0)

    def fill_zero(i, _):
        zero_v[pl.ds(i * 16, 16)] = jnp.zeros((16,), jnp.float32)
        return 0

    lax.fori_loop(0, _DSL // 16, fill_zero, 0)

    def cbase(t):
        return jnp.where(t < DFULL, wE + t * CH, E + wE + (t - DFULL) * CH)

    def start_idx(t, buf, sem):
        pltpu.async_copy(dst2.at[pl.ds(cbase(t), CH)], buf, sem)

    def wait_idx(t, buf, sem):
        pltpu.make_async_copy(dst2.at[pl.ds(cbase(t), CH)], buf, sem).wait()

    start_idx(0, idx0, semi0)
    pltpu.sync_copy(zero_v, deg_sh.at[pl.ds(s * _DSL, _DSL)])
    plsc.subcore_barrier()

    def body(u, _):
        t0 = 2 * u
        start_idx(t0 + 1, idx1, semi1)
        wait_idx(t0, idx0, semi0)
        pltpu.sync_copy(ones_v, deg_sh.at[idx0], add=True)

        @pl.when(u < _DCH // 2 - 1)
        def _():
            start_idx(t0 + 2, idx0, semi0)

        wait_idx(t0 + 1, idx1, semi1)
        pltpu.sync_copy(ones_v, deg_sh.at[idx1], add=True)
        return 0

    lax.fori_loop(0, _DCH // 2, body, 0)

    pltpu.sync_copy(dst2.at[pl.ds(wE + DFULL * CH, DTAIL)], idxtA)
    pltpu.sync_copy(dst2.at[pl.ds(E + wE + DFULL * CH, DTAIL)], idxtB)
    pltpu.sync_copy(ones_v.at[pl.ds(0, DTAIL)], deg_sh.at[idxtA], add=True)
    pltpu.sync_copy(ones_v.at[pl.ds(0, DTAIL)], deg_sh.at[idxtB], add=True)

    plsc.subcore_barrier()
    pltpu.sync_copy(deg_sh.at[pl.ds(s * _DSL, _DSL)],
                    out.at[pl.ds(c * 2 * ND + s * _DSL, _DSL)])


_NBUF = 3
_URND = NFULL // _NBUF


def _scatter_body(gA, gB, srcx, dst, out, acc_sh, sidx, didx, rows, sidxt,
                  didxt, semg, sems):
    c = lax.axis_index("c")
    s = lax.axis_index("s")
    r0 = s * RPT
    base0 = s * EPS

    for p, g2 in enumerate((gA, gB)):
        cE = p * 2 * E + c * E
        dE = p * E
        oB = p * 2 * N

        def load_idx(b, j):
            pltpu.sync_copy(srcx.at[pl.ds(cE + b, CH)], sidx[j])
            pltpu.sync_copy(dst.at[pl.ds(dE + b, CH)], didx[j])

        def start_gather(j):
            pltpu.async_copy(g2.at[sidx[j]], rows[j], semg[j])

        def wait_gather(j):
            pltpu.make_async_copy(g2.at[sidx[j]], rows[j], semg[j]).wait()

        def start_scatter(j):
            pltpu.async_copy(rows[j], acc_sh.at[didx[j]], sems[j], add=True)

        def wait_scatter(j):
            pltpu.make_async_copy(rows[j], acc_sh.at[didx[j]], sems[j]).wait()

        for j in range(_NBUF - 1):
            load_idx(base0 + j * CH, j)
            start_gather(j)

        pltpu.sync_copy(g2.at[pl.ds(c * N + r0, RPT)],
                        acc_sh.at[pl.ds(r0, RPT)])

        @pl.when(s == NS - 1)
        def _():
            pltpu.sync_copy(g2.at[pl.ds(c * N + NS * RPT, RREM)],
                            acc_sh.at[pl.ds(NS * RPT, RREM)])

        plsc.subcore_barrier()

        def body(u, _):
            for j in range(_NBUF):
                t_is_0 = (u == 0) & (j == 0)
                last_ok = jnp.logical_or(u < _URND - 1, j == 0)
                wait_gather(j)
                start_scatter(j)
                b2 = (j + _NBUF - 1) % _NBUF

                @pl.when(jnp.logical_not(t_is_0))
                def _():
                    wait_scatter(b2)

                @pl.when(last_ok)
                def _():
                    load_idx(base0 + (_NBUF * u + j + _NBUF - 1) * CH, b2)
                    start_gather(b2)
            return 0

        lax.fori_loop(0, _URND, body, 0)
        wait_scatter((NFULL - 1) % _NBUF)

        bt = base0 + NFULL * CH
        pltpu.sync_copy(srcx.at[pl.ds(cE + bt, TAIL)], sidxt)
        pltpu.sync_copy(dst.at[pl.ds(dE + bt, TAIL)], didxt)
        pltpu.async_copy(g2.at[sidxt], rows[0].at[pl.ds(0, TAIL)],
                         semg[0]).wait()
        pltpu.sync_copy(rows[0].at[pl.ds(0, TAIL)], acc_sh.at[didxt],
                        add=True)

        plsc.subcore_barrier()
        pltpu.sync_copy(acc_sh.at[pl.ds(r0, RPT)],
                        out.at[pl.ds(oB + c * N + r0, RPT)])

        @pl.when(s == NS - 1)
        def _():
            pltpu.sync_copy(acc_sh.at[pl.ds(NS * RPT, RREM)],
                            out.at[pl.ds(oB + c * N + NS * RPT, RREM)])

        if p == 0:
            plsc.subcore_barrier()


@functools.cache
def _deg_kernel_fn():
    return pl.kernel(
        _deg_body,
        out_type=jax.ShapeDtypeStruct((NC * 2 * ND,), jnp.float32),
        mesh=_mesh(),
        scratch_types=[
            pltpu.VMEM_SHARED((2 * ND,), jnp.float32),
            pltpu.VMEM((CH,), jnp.int32),
            pltpu.VMEM((CH,), jnp.int32),
            pltpu.VMEM((DTAIL,), jnp.int32),
            pltpu.VMEM((DTAIL,), jnp.int32),
            pltpu.VMEM((CH,), jnp.float32),
            pltpu.VMEM((_DSL,), jnp.float32),
            pltpu.SemaphoreType.DMA,
            pltpu.SemaphoreType.DMA,
        ],
    )


def _deg_kernel(dst2):
    return _deg_kernel_fn()(dst2)


@functools.cache
def _scatter_kernel_fn():
    return pl.kernel(
        _scatter_body,
        out_type=jax.ShapeDtypeStruct((4 * N, HALF), jnp.float32),
        mesh=_mesh(),
        scratch_types=[
            pltpu.VMEM_SHARED((N, HALF), jnp.float32),
            [pltpu.VMEM((CH,), jnp.int32)] * _NBUF,
            [pltpu.VMEM((CH,), jnp.int32)] * _NBUF,
            [pltpu.VMEM((CH, HALF), jnp.float32)] * _NBUF,
            pltpu.VMEM((TAIL,), jnp.int32),
            pltpu.VMEM((TAIL,), jnp.int32),
            [pltpu.SemaphoreType.DMA] * _NBUF,
            [pltpu.SemaphoreType.DMA] * _NBUF,
        ],
    )


def _mm_scale_body(x_ref, w_ref, dv_ref, o_ref):
    o_ref[...] = jnp.dot(x_ref[...], w_ref[...],
                         preferred_element_type=jnp.float32) * dv_ref[...]


def _mm_scale(xg, W, dv):
    return pl.pallas_call(
        _mm_scale_body,
        grid=(NB, 2),
        in_specs=[
            pl.BlockSpec((RB, D), lambda i, c: (i, 0)),
            pl.BlockSpec((D, HALF), lambda i, c: (0, c)),
            pl.BlockSpec((RB, 1), lambda i, c: (i, 0)),
        ],
        out_specs=pl.BlockSpec((RB, HALF), lambda i, c: (c * NB + i, 0)),
        out_shape=jax.ShapeDtypeStruct((2 * N, HALF), jnp.float32),
    )(xg, W, dv)


def _combine_mm_body(alo_ref, ahi_ref, dv_ref, b_ref, w_ref, o_ref):
    dv = dv_ref[...]
    hlo = jnp.maximum(dv * alo_ref[...] + b_ref[0:1, 0:HALF], 0.0)
    hhi = jnp.maximum(dv * ahi_ref[...] + b_ref[0:1, HALF:D], 0.0)
    o_ref[...] = (jnp.dot(hlo, w_ref[0:HALF, :],
                          preferred_element_type=jnp.float32)
                  + jnp.dot(hhi, w_ref[HALF:D, :],
                            preferred_element_type=jnp.float32)) * dv


def _combine_mm(acc4, dv, b, W, half):
    base = half * 2 * NB
    return pl.pallas_call(
        _combine_mm_body,
        grid=(NB, 2),
        in_specs=[
            pl.BlockSpec((RB, HALF), lambda i, c: (base + i, 0)),
            pl.BlockSpec((RB, HALF), lambda i, c: (base + NB + i, 0)),
            pl.BlockSpec((RB, 1), lambda i, c: (i, 0)),
            pl.BlockSpec((1, D), lambda i, c: (0, 0)),
            pl.BlockSpec((D, HALF), lambda i, c: (0, c)),
        ],
        out_specs=pl.BlockSpec((RB, HALF), lambda i, c: (c * NB + i, 0)),
        out_shape=jax.ShapeDtypeStruct((2 * N, HALF), jnp.float32),
    )(acc4, acc4, dv, b, W)


def _final2_body(aA_ref, aB_ref, dvA_ref, dvB_ref, b_ref, zA_ref, zB_ref,
                 cs_ref):
    zA = dvA_ref[...] * aA_ref[...] + b_ref[...]
    zA_ref[...] = zA
    zB_ref[...] = dvB_ref[...] * aB_ref[...] + b_ref[...]

    @pl.when(pl.program_id(0) == 0)
    def _():
        cs_ref[...] = jnp.zeros_like(cs_ref)

    cs_ref[...] += jnp.sum(zA, axis=0, keepdims=True)


def _final2(acc4, dvA, dvB, b):
    return pl.pallas_call(
        _final2_body,
        grid=(NB, 2),
        in_specs=[
            pl.BlockSpec((RB, HALF), lambda i, c: (c * NB + i, 0)),
            pl.BlockSpec((RB, HALF), lambda i, c: (2 * NB + c * NB + i, 0)),
            pl.BlockSpec((RB, 1), lambda i, c: (i, 0)),
            pl.BlockSpec((RB, 1), lambda i, c: (i, 0)),
            pl.BlockSpec((1, HALF), lambda i, c: (0, c)),
        ],
        out_specs=[
            pl.BlockSpec((RB, HALF), lambda i, c: (i, c)),
            pl.BlockSpec((RB, HALF), lambda i, c: (i, c)),
            pl.BlockSpec((1, HALF), lambda i, c: (0, c)),
        ],
        out_shape=[
            jax.ShapeDtypeStruct((N, D), jnp.float32),
            jax.ShapeDtypeStruct((N, D), jnp.float32),
            jax.ShapeDtypeStruct((1, D), jnp.float32),
        ],
    )(acc4, acc4, dvA, dvB, b)


def _scores_body(z_ref, zc_ref, cs_ref, p_ref, n_ref):
    sm = cs_ref[...] * (1.0 / N)
    dn = (((1,), (1,)), ((), ()))
    p = lax.dot_general(z_ref[...], sm, dn, preferred_element_type=jnp.float32)
    n = lax.dot_general(zc_ref[...], sm, dn, preferred_element_type=jnp.float32)
    p_ref[...] = 1.0 / (1.0 + jnp.exp(-p))
    n_ref[...] = 1.0 / (1.0 + jnp.exp(-n))


def _scores(z, z_c, colsum):
    return pl.pallas_call(
        _scores_body,
        grid=(NB,),
        in_specs=[
            pl.BlockSpec((RB, D), lambda i: (i, 0)),
            pl.BlockSpec((RB, D), lambda i: (i, 0)),
            pl.BlockSpec((1, D), lambda i: (0, 0)),
        ],
        out_specs=[
            pl.BlockSpec((RB, 1), lambda i: (i, 0)),
            pl.BlockSpec((RB, 1), lambda i: (i, 0)),
        ],
        out_shape=[
            jax.ShapeDtypeStruct((N, 1), jnp.float32),
            jax.ShapeDtypeStruct((N, 1), jnp.float32),
        ],
    )(z, z_c, colsum)


def kernel(x, edge_index, batch, x_corrupted, edge_index_corrupted,
           batch_corrupted, W1, b1, W2, b2):
    src, dst = edge_index[0], edge_index[1]
    src_c, dst_c = edge_index_corrupted[0], edge_index_corrupted[1]

    dst2 = jnp.concatenate([dst, dst_c + ND])
    degp = _deg_kernel(dst2).reshape(NC, 2, ND)
    deg = degp[0, :, :N] + degp[1, :, :N] + 1.0
    dinv = lax.rsqrt(deg)

    b1r = b1.reshape(1, D)
    b2r = b2.reshape(1, D)
    dvA = dinv[0][:, None]
    dvB = dinv[1][:, None]

    srcx = jnp.concatenate([src, src + N, src_c, src_c + N])
    dstall = jnp.concatenate([dst, dst_c])

    gA1 = _mm_scale(x, W1, dvA)
    gB1 = _mm_scale(x_corrupted, W1, dvB)
    acc1 = _scatter_kernel_fn()(gA1, gB1, srcx, dstall)
    gA2 = _combine_mm(acc1, dvA, b1r, W2, 0)
    gB2 = _combine_mm(acc1, dvB, b1r, W2, 1)
    acc2 = _scatter_kernel_fn()(gA2, gB2, srcx, dstall)
    z, z_c, colsum = _final2(acc2, dvA, dvB, b2r)
    pos, neg = _scores(z, z_c, colsum)
    return pos[:, 0], neg[:, 0], z


# speedup vs baseline: 1.0864x; 1.0864x over previous
import functools

import jax
import jax.numpy as jnp
from jax import lax
from jax.experimental import pallas as pl
from jax.experimental.pallas import tpu as pltpu
from jax.experimental.pallas import tpu_sc as plsc

N = 10000
D = 256
E = 160000
HALF = 128
NC = 2
NS = 16
NW = NC * NS

ND = 10240
NDS = ND // NS
EPW = E // NW
CH = 128
DFULL = EPW // CH
DTAIL = EPW - DFULL * CH

EPS = E // NS
NFULL = EPS // CH
TAIL = EPS - NFULL * CH
RPT = 624
RREM = N - NS * RPT

RB = 1000
NB = N // RB

@functools.cache
def _mesh():
    return plsc.VectorSubcoreMesh(core_axis_name="c", subcore_axis_name="s",
                                  num_cores=NC, num_subcores=NS)


_DCH = 2 * DFULL
_DSL = 2 * ND // NS


def _deg_body(dst2, out, deg_sh, idx0, idx1, idxtA, idxtB, ones_v, zero_v,
              semi0, semi1):
    c = lax.axis_index("c")
    s = lax.axis_index("s")
    w = s * NC + c
    wE = w * EPW

    def fill_ones(i, _):
        ones_v[pl.ds(i * 16, 16)] = jnp.full((16,), 1.0, jnp.float32)
        return 0

    lax.fori_loop(0, CH // 16, fill_ones, 0)

    def fill_zero(i, _):
        zero_v[pl.ds(i * 16, 16)] = jnp.zeros((16,), jnp.float32)
        return 0

    lax.fori_loop(0, _DSL // 16, fill_zero, 0)

    def cbase(t):
        return jnp.where(t < DFULL, wE + t * CH, E + wE + (t - DFULL) * CH)

    def start_idx(t, buf, sem):
        pltpu.async_copy(dst2.at[pl.ds(cbase(t), CH)], buf, sem)

    def wait_idx(t, buf, sem):
        pltpu.make_async_copy(dst2.at[pl.ds(cbase(t), CH)], buf, sem).wait()

    start_idx(0, idx0, semi0)
    pltpu.sync_copy(zero_v, deg_sh.at[pl.ds(s * _DSL, _DSL)])
    plsc.subcore_barrier()

    def body(u, _):
        t0 = 2 * u
        start_idx(t0 + 1, idx1, semi1)
        wait_idx(t0, idx0, semi0)
        pltpu.sync_copy(ones_v, deg_sh.at[idx0], add=True)

        @pl.when(u < _DCH // 2 - 1)
        def _():
            start_idx(t0 + 2, idx0, semi0)

        wait_idx(t0 + 1, idx1, semi1)
        pltpu.sync_copy(ones_v, deg_sh.at[idx1], add=True)
        return 0

    lax.fori_loop(0, _DCH // 2, body, 0)

    pltpu.sync_copy(dst2.at[pl.ds(wE + DFULL * CH, DTAIL)], idxtA)
    pltpu.sync_copy(dst2.at[pl.ds(E + wE + DFULL * CH, DTAIL)], idxtB)
    pltpu.sync_copy(ones_v.at[pl.ds(0, DTAIL)], deg_sh.at[idxtA], add=True)
    pltpu.sync_copy(ones_v.at[pl.ds(0, DTAIL)], deg_sh.at[idxtB], add=True)

    plsc.subcore_barrier()
    pltpu.sync_copy(deg_sh.at[pl.ds(s * _DSL, _DSL)],
                    out.at[pl.ds(c * 2 * ND + s * _DSL, _DSL)])


_NBUF = 3
_URND = NFULL // _NBUF


def _scatter_body(g2, srcx, dst, out, acc_sh, sidx, didx, rows, sidxt,
                  didxt, semg, sems):
    c = lax.axis_index("c")
    s = lax.axis_index("s")
    r0 = s * RPT
    base0 = s * EPS
    cE = c * E

    def load_idx(b, j):
        pltpu.sync_copy(srcx.at[pl.ds(cE + b, CH)], sidx[j])
        pltpu.sync_copy(dst.at[pl.ds(b, CH)], didx[j])

    def start_gather(j):
        pltpu.async_copy(g2.at[sidx[j]], rows[j], semg[j])

    def wait_gather(j):
        pltpu.make_async_copy(g2.at[sidx[j]], rows[j], semg[j]).wait()

    def start_scatter(j):
        pltpu.async_copy(rows[j], acc_sh.at[didx[j]], sems[j], add=True)

    def wait_scatter(j):
        pltpu.make_async_copy(rows[j], acc_sh.at[didx[j]], sems[j]).wait()

    for j in range(_NBUF - 1):
        load_idx(base0 + j * CH, j)
        start_gather(j)

    pltpu.sync_copy(g2.at[pl.ds(c * N + r0, RPT)], acc_sh.at[pl.ds(r0, RPT)])

    @pl.when(s == NS - 1)
    def _():
        pltpu.sync_copy(g2.at[pl.ds(c * N + NS * RPT, RREM)],
                        acc_sh.at[pl.ds(NS * RPT, RREM)])

    plsc.subcore_barrier()

    def body(u, _):
        for j in range(_NBUF):
            t_is_0 = (u == 0) & (j == 0)
            last_ok = jnp.logical_or(u < _URND - 1, j == 0)
            wait_gather(j)
            start_scatter(j)
            b2 = (j + _NBUF - 1) % _NBUF

            @pl.when(jnp.logical_not(t_is_0))
            def _():
                wait_scatter(b2)

            @pl.when(last_ok)
            def _():
                load_idx(base0 + (_NBUF * u + j + _NBUF - 1) * CH, b2)
                start_gather(b2)
        return 0

    lax.fori_loop(0, _URND, body, 0)
    wait_scatter((NFULL - 1) % _NBUF)

    bt = base0 + NFULL * CH
    pltpu.sync_copy(srcx.at[pl.ds(cE + bt, TAIL)], sidxt)
    pltpu.sync_copy(dst.at[pl.ds(bt, TAIL)], didxt)
    pltpu.async_copy(g2.at[sidxt], rows[0].at[pl.ds(0, TAIL)], semg[0]).wait()
    pltpu.sync_copy(rows[0].at[pl.ds(0, TAIL)], acc_sh.at[didxt], add=True)

    plsc.subcore_barrier()
    pltpu.sync_copy(acc_sh.at[pl.ds(r0, RPT)], out.at[pl.ds(c * N + r0, RPT)])

    @pl.when(s == NS - 1)
    def _():
        pltpu.sync_copy(acc_sh.at[pl.ds(NS * RPT, RREM)],
                        out.at[pl.ds(c * N + NS * RPT, RREM)])


@functools.cache
def _deg_kernel_fn():
    return pl.kernel(
        _deg_body,
        out_type=jax.ShapeDtypeStruct((NC * 2 * ND,), jnp.float32),
        mesh=_mesh(),
        scratch_types=[
            pltpu.VMEM_SHARED((2 * ND,), jnp.float32),
            pltpu.VMEM((CH,), jnp.int32),
            pltpu.VMEM((CH,), jnp.int32),
            pltpu.VMEM((DTAIL,), jnp.int32),
            pltpu.VMEM((DTAIL,), jnp.int32),
            pltpu.VMEM((CH,), jnp.float32),
            pltpu.VMEM((_DSL,), jnp.float32),
            pltpu.SemaphoreType.DMA,
            pltpu.SemaphoreType.DMA,
        ],
    )


def _deg_kernel(dst2):
    return _deg_kernel_fn()(dst2)


def _scatter_kernel(g2, srcx, dst):
    return _scatter_kernel_fn()(g2, srcx, dst)


@functools.cache
def _scatter_kernel_fn():
    return pl.kernel(
        _scatter_body,
        out_type=jax.ShapeDtypeStruct((2 * N, HALF), jnp.float32),
        mesh=_mesh(),
        scratch_types=[
            pltpu.VMEM_SHARED((N, HALF), jnp.float32),
            [pltpu.VMEM((CH,), jnp.int32)] * _NBUF,
            [pltpu.VMEM((CH,), jnp.int32)] * _NBUF,
            [pltpu.VMEM((CH, HALF), jnp.float32)] * _NBUF,
            pltpu.VMEM((TAIL,), jnp.int32),
            pltpu.VMEM((TAIL,), jnp.int32),
            [pltpu.SemaphoreType.DMA] * _NBUF,
            [pltpu.SemaphoreType.DMA] * _NBUF,
        ],
    )


def _mm_scale_body(x_ref, w_ref, dv_ref, o_ref):
    o_ref[...] = jnp.dot(x_ref[...], w_ref[...],
                         preferred_element_type=jnp.float32) * dv_ref[...]


def _mm_scale(xg, W, dv):
    return pl.pallas_call(
        _mm_scale_body,
        grid=(NB, 2),
        in_specs=[
            pl.BlockSpec((RB, D), lambda i, c: (i, 0)),
            pl.BlockSpec((D, HALF), lambda i, c: (0, c)),
            pl.BlockSpec((RB, 1), lambda i, c: (i, 0)),
        ],
        out_specs=pl.BlockSpec((RB, HALF), lambda i, c: (c * NB + i, 0)),
        out_shape=jax.ShapeDtypeStruct((2 * N, HALF), jnp.float32),
    )(xg, W, dv)


def _combine_mm_body(alo_ref, ahi_ref, dv_ref, b_ref, w_ref, o_ref):
    dv = dv_ref[...]
    hlo = jnp.maximum(dv * alo_ref[...] + b_ref[0:1, 0:HALF], 0.0)
    hhi = jnp.maximum(dv * ahi_ref[...] + b_ref[0:1, HALF:D], 0.0)
    o_ref[...] = (jnp.dot(hlo, w_ref[0:HALF, :],
                          preferred_element_type=jnp.float32)
                  + jnp.dot(hhi, w_ref[HALF:D, :],
                            preferred_element_type=jnp.float32)) * dv


def _combine_mm(acc, dv, b, W):
    return pl.pallas_call(
        _combine_mm_body,
        grid=(NB, 2),
        in_specs=[
            pl.BlockSpec((RB, HALF), lambda i, c: (i, 0)),
            pl.BlockSpec((RB, HALF), lambda i, c: (NB + i, 0)),
            pl.BlockSpec((RB, 1), lambda i, c: (i, 0)),
            pl.BlockSpec((1, D), lambda i, c: (0, 0)),
            pl.BlockSpec((D, HALF), lambda i, c: (0, c)),
        ],
        out_specs=pl.BlockSpec((RB, HALF), lambda i, c: (c * NB + i, 0)),
        out_shape=jax.ShapeDtypeStruct((2 * N, HALF), jnp.float32),
    )(acc, acc, dv, b, W)


def _final2_body(aA_ref, aB_ref, dvA_ref, dvB_ref, b_ref, zA_ref, zB_ref,
                 cs_ref):
    zA = dvA_ref[...] * aA_ref[...] + b_ref[...]
    zA_ref[...] = zA
    zB_ref[...] = dvB_ref[...] * aB_ref[...] + b_ref[...]

    @pl.when(pl.program_id(0) == 0)
    def _():
        cs_ref[...] = jnp.zeros_like(cs_ref)

    cs_ref[...] += jnp.sum(zA, axis=0, keepdims=True)


def _final2(accA, accB, dvA, dvB, b):
    return pl.pallas_call(
        _final2_body,
        grid=(NB, 2),
        in_specs=[
            pl.BlockSpec((RB, HALF), lambda i, c: (c * NB + i, 0)),
            pl.BlockSpec((RB, HALF), lambda i, c: (c * NB + i, 0)),
            pl.BlockSpec((RB, 1), lambda i, c: (i, 0)),
            pl.BlockSpec((RB, 1), lambda i, c: (i, 0)),
            pl.BlockSpec((1, HALF), lambda i, c: (0, c)),
        ],
        out_specs=[
            pl.BlockSpec((RB, HALF), lambda i, c: (i, c)),
            pl.BlockSpec((RB, HALF), lambda i, c: (i, c)),
            pl.BlockSpec((1, HALF), lambda i, c: (0, c)),
        ],
        out_shape=[
            jax.ShapeDtypeStruct((N, D), jnp.float32),
            jax.ShapeDtypeStruct((N, D), jnp.float32),
            jax.ShapeDtypeStruct((1, D), jnp.float32),
        ],
    )(accA, accB, dvA, dvB, b)


def _scores_body(z_ref, zc_ref, cs_ref, p_ref, n_ref):
    sm = cs_ref[...] * (1.0 / N)
    dn = (((1,), (1,)), ((), ()))
    p = lax.dot_general(z_ref[...], sm, dn, preferred_element_type=jnp.float32)
    n = lax.dot_general(zc_ref[...], sm, dn, preferred_element_type=jnp.float32)
    p_ref[...] = 1.0 / (1.0 + jnp.exp(-p))
    n_ref[...] = 1.0 / (1.0 + jnp.exp(-n))


def _scores(z, z_c, colsum):
    return pl.pallas_call(
        _scores_body,
        grid=(NB,),
        in_specs=[
            pl.BlockSpec((RB, D), lambda i: (i, 0)),
            pl.BlockSpec((RB, D), lambda i: (i, 0)),
            pl.BlockSpec((1, D), lambda i: (0, 0)),
        ],
        out_specs=[
            pl.BlockSpec((RB, 1), lambda i: (i, 0)),
            pl.BlockSpec((RB, 1), lambda i: (i, 0)),
        ],
        out_shape=[
            jax.ShapeDtypeStruct((N, 1), jnp.float32),
            jax.ShapeDtypeStruct((N, 1), jnp.float32),
        ],
    )(z, z_c, colsum)


def kernel(x, edge_index, batch, x_corrupted, edge_index_corrupted,
           batch_corrupted, W1, b1, W2, b2):
    src, dst = edge_index[0], edge_index[1]
    src_c, dst_c = edge_index_corrupted[0], edge_index_corrupted[1]

    dst2 = jnp.concatenate([dst, dst_c + ND])
    degp = _deg_kernel(dst2).reshape(NC, 2, ND)
    deg = degp[0, :, :N] + degp[1, :, :N] + 1.0
    dinv = lax.rsqrt(deg)

    b1r = b1.reshape(1, D)
    b2r = b2.reshape(1, D)
    dvA = dinv[0][:, None]
    dvB = dinv[1][:, None]

    srcxA = jnp.concatenate([src, src + N])
    srcxB = jnp.concatenate([src_c, src_c + N])

    gA1 = _mm_scale(x, W1, dvA)
    accA1 = _scatter_kernel(gA1, srcxA, dst)
    gB1 = _mm_scale(x_corrupted, W1, dvB)
    accB1 = _scatter_kernel(gB1, srcxB, dst_c)
    gA2 = _combine_mm(accA1, dvA, b1r, W2)
    accA2 = _scatter_kernel(gA2, srcxA, dst)
    gB2 = _combine_mm(accB1, dvB, b1r, W2)
    accB2 = _scatter_kernel(gB2, srcxB, dst_c)
    z, z_c, colsum = _final2(accA2, accB2, dvA, dvB, b2r)
    pos, neg = _scores(z, z_c, colsum)
    return pos[:, 0], neg[:, 0], z


# speedup vs baseline: 1.0941x; 1.0071x over previous
import functools

import jax
import jax.numpy as jnp
from jax import lax
from jax.experimental import pallas as pl
from jax.experimental.pallas import tpu as pltpu
from jax.experimental.pallas import tpu_sc as plsc

N = 10000
D = 256
E = 160000
HALF = 128
NC = 2
NS = 16
NW = NC * NS

ND = 10240
NDS = ND // NS
EPW = E // NW
CH = 128
DFULL = EPW // CH
DTAIL = EPW - DFULL * CH

EPS = E // NS
NFULL = EPS // CH
TAIL = EPS - NFULL * CH
RPT = 624
RREM = N - NS * RPT

RB = 1000
NB = N // RB

@functools.cache
def _mesh():
    return plsc.VectorSubcoreMesh(core_axis_name="c", subcore_axis_name="s",
                                  num_cores=NC, num_subcores=NS)


_DCH = 2 * DFULL
_DSL = 2 * ND // NS


def _deg_body(dst2, out, deg_sh, idx0, idx1, idxtA, idxtB, ones_v, zero_v,
              semi0, semi1):
    c = lax.axis_index("c")
    s = lax.axis_index("s")
    w = s * NC + c
    wE = w * EPW

    def fill_ones(i, _):
        ones_v[pl.ds(i * 16, 16)] = jnp.full((16,), 1.0, jnp.float32)
        return 0

    lax.fori_loop(0, CH // 16, fill_ones, 0)

    def fill_zero(i, _):
        zero_v[pl.ds(i * 16, 16)] = jnp.zeros((16,), jnp.float32)
        return 0

    lax.fori_loop(0, _DSL // 16, fill_zero, 0)

    def cbase(t):
        return jnp.where(t < DFULL, wE + t * CH, E + wE + (t - DFULL) * CH)

    def start_idx(t, buf, sem):
        pltpu.async_copy(dst2.at[pl.ds(cbase(t), CH)], buf, sem)

    def wait_idx(t, buf, sem):
        pltpu.make_async_copy(dst2.at[pl.ds(cbase(t), CH)], buf, sem).wait()

    start_idx(0, idx0, semi0)
    pltpu.sync_copy(zero_v, deg_sh.at[pl.ds(s * _DSL, _DSL)])
    plsc.subcore_barrier()

    def body(u, _):
        t0 = 2 * u
        start_idx(t0 + 1, idx1, semi1)
        wait_idx(t0, idx0, semi0)
        pltpu.sync_copy(ones_v, deg_sh.at[idx0], add=True)

        @pl.when(u < _DCH // 2 - 1)
        def _():
            start_idx(t0 + 2, idx0, semi0)

        wait_idx(t0 + 1, idx1, semi1)
        pltpu.sync_copy(ones_v, deg_sh.at[idx1], add=True)
        return 0

    lax.fori_loop(0, _DCH // 2, body, 0)

    pltpu.sync_copy(dst2.at[pl.ds(wE + DFULL * CH, DTAIL)], idxtA)
    pltpu.sync_copy(dst2.at[pl.ds(E + wE + DFULL * CH, DTAIL)], idxtB)
    pltpu.sync_copy(ones_v.at[pl.ds(0, DTAIL)], deg_sh.at[idxtA], add=True)
    pltpu.sync_copy(ones_v.at[pl.ds(0, DTAIL)], deg_sh.at[idxtB], add=True)

    plsc.subcore_barrier()
    pltpu.sync_copy(deg_sh.at[pl.ds(s * _DSL, _DSL)],
                    out.at[pl.ds(c * 2 * ND + s * _DSL, _DSL)])


_NBUF = 3
_URND = NFULL // _NBUF


def _scatter_body(g2, srcx, dst, out, acc_sh, sidx, didx, rows, sidxt,
                  didxt, semg, sems):
    c = lax.axis_index("c")
    s = lax.axis_index("s")
    r0 = s * RPT
    base0 = s * EPS
    cE = c * E

    def load_idx(b, j):
        pltpu.sync_copy(srcx.at[pl.ds(cE + b, CH)], sidx[j])
        pltpu.sync_copy(dst.at[pl.ds(b, CH)], didx[j])

    def start_gather(j):
        pltpu.async_copy(g2.at[sidx[j]], rows[j], semg[j])

    def wait_gather(j):
        pltpu.make_async_copy(g2.at[sidx[j]], rows[j], semg[j]).wait()

    def start_scatter(j):
        pltpu.async_copy(rows[j], acc_sh.at[didx[j]], sems[j], add=True)

    def wait_scatter(j):
        pltpu.make_async_copy(rows[j], acc_sh.at[didx[j]], sems[j]).wait()

    for j in range(_NBUF - 1):
        load_idx(base0 + j * CH, j)
        start_gather(j)

    pltpu.sync_copy(g2.at[pl.ds(c * N + r0, RPT)], acc_sh.at[pl.ds(r0, RPT)])

    @pl.when(s == NS - 1)
    def _():
        pltpu.sync_copy(g2.at[pl.ds(c * N + NS * RPT, RREM)],
                        acc_sh.at[pl.ds(NS * RPT, RREM)])

    plsc.subcore_barrier()

    def body(u, _):
        for j in range(_NBUF):
            t_is_0 = (u == 0) & (j == 0)
            last_ok = jnp.logical_or(u < _URND - 1, j == 0)
            wait_gather(j)
            start_scatter(j)
            b2 = (j + _NBUF - 1) % _NBUF

            @pl.when(jnp.logical_not(t_is_0))
            def _():
                wait_scatter(b2)

            @pl.when(last_ok)
            def _():
                load_idx(base0 + (_NBUF * u + j + _NBUF - 1) * CH, b2)
                start_gather(b2)
        return 0

    lax.fori_loop(0, _URND, body, 0)
    wait_scatter((NFULL - 1) % _NBUF)

    bt = base0 + NFULL * CH
    pltpu.sync_copy(srcx.at[pl.ds(cE + bt, TAIL)], sidxt)
    pltpu.sync_copy(dst.at[pl.ds(bt, TAIL)], didxt)
    pltpu.async_copy(g2.at[sidxt], rows[0].at[pl.ds(0, TAIL)], semg[0]).wait()
    pltpu.sync_copy(rows[0].at[pl.ds(0, TAIL)], acc_sh.at[didxt], add=True)

    plsc.subcore_barrier()
    pltpu.sync_copy(acc_sh.at[pl.ds(r0, RPT)], out.at[pl.ds(c * N + r0, RPT)])

    @pl.when(s == NS - 1)
    def _():
        pltpu.sync_copy(acc_sh.at[pl.ds(NS * RPT, RREM)],
                        out.at[pl.ds(c * N + NS * RPT, RREM)])


@functools.cache
def _deg_kernel_fn():
    return pl.kernel(
        _deg_body,
        out_type=jax.ShapeDtypeStruct((NC * 2 * ND,), jnp.float32),
        mesh=_mesh(),
        scratch_types=[
            pltpu.VMEM_SHARED((2 * ND,), jnp.float32),
            pltpu.VMEM((CH,), jnp.int32),
            pltpu.VMEM((CH,), jnp.int32),
            pltpu.VMEM((DTAIL,), jnp.int32),
            pltpu.VMEM((DTAIL,), jnp.int32),
            pltpu.VMEM((CH,), jnp.float32),
            pltpu.VMEM((_DSL,), jnp.float32),
            pltpu.SemaphoreType.DMA,
            pltpu.SemaphoreType.DMA,
        ],
    )


def _deg_kernel(dst2):
    return _deg_kernel_fn()(dst2)


def _scatter_kernel(g2, srcx, dst):
    return _scatter_kernel_fn()(g2, srcx, dst)


@functools.cache
def _scatter_kernel_fn():
    return pl.kernel(
        _scatter_body,
        out_type=jax.ShapeDtypeStruct((2 * N, HALF), jnp.float32),
        mesh=_mesh(),
        scratch_types=[
            pltpu.VMEM_SHARED((N, HALF), jnp.float32),
            [pltpu.VMEM((CH,), jnp.int32)] * _NBUF,
            [pltpu.VMEM((CH,), jnp.int32)] * _NBUF,
            [pltpu.VMEM((CH, HALF), jnp.float32)] * _NBUF,
            pltpu.VMEM((TAIL,), jnp.int32),
            pltpu.VMEM((TAIL,), jnp.int32),
            [pltpu.SemaphoreType.DMA] * _NBUF,
            [pltpu.SemaphoreType.DMA] * _NBUF,
        ],
    )


def _mm_scale_body(x_ref, w_ref, dv_ref, o_ref):
    o_ref[...] = jnp.dot(x_ref[...], w_ref[...],
                         preferred_element_type=jnp.float32) * dv_ref[...]


def _mm_scale(xg, W, dv):
    return pl.pallas_call(
        _mm_scale_body,
        grid=(NB, 2),
        in_specs=[
            pl.BlockSpec((RB, D), lambda i, c: (i, 0)),
            pl.BlockSpec((D, HALF), lambda i, c: (0, c)),
            pl.BlockSpec((RB, 1), lambda i, c: (i, 0)),
        ],
        out_specs=pl.BlockSpec((RB, HALF), lambda i, c: (c * NB + i, 0)),
        out_shape=jax.ShapeDtypeStruct((2 * N, HALF), jnp.float32),
    )(xg, W, dv)


def _combine_mm_body(alo_ref, ahi_ref, dv_ref, b_ref, w_ref, o_ref):
    dv = dv_ref[...]
    hlo = jnp.maximum(dv * alo_ref[...] + b_ref[0:1, 0:HALF], 0.0)
    hhi = jnp.maximum(dv * ahi_ref[...] + b_ref[0:1, HALF:D], 0.0)
    o_ref[...] = (jnp.dot(hlo, w_ref[0:HALF, :],
                          preferred_element_type=jnp.float32)
                  + jnp.dot(hhi, w_ref[HALF:D, :],
                            preferred_element_type=jnp.float32)) * dv


def _combine_mm(acc, dv, b, W):
    return pl.pallas_call(
        _combine_mm_body,
        grid=(NB, 2),
        in_specs=[
            pl.BlockSpec((RB, HALF), lambda i, c: (i, 0)),
            pl.BlockSpec((RB, HALF), lambda i, c: (NB + i, 0)),
            pl.BlockSpec((RB, 1), lambda i, c: (i, 0)),
            pl.BlockSpec((1, D), lambda i, c: (0, 0)),
            pl.BlockSpec((D, HALF), lambda i, c: (0, c)),
        ],
        out_specs=pl.BlockSpec((RB, HALF), lambda i, c: (c * NB + i, 0)),
        out_shape=jax.ShapeDtypeStruct((2 * N, HALF), jnp.float32),
    )(acc, acc, dv, b, W)


def _finalA_body(alo_ref, ahi_ref, dv_ref, b_ref, z_ref, cs_ref):
    dv = dv_ref[...]
    zlo = dv * alo_ref[...] + b_ref[0:1, 0:HALF]
    zhi = dv * ahi_ref[...] + b_ref[0:1, HALF:D]
    z = jnp.concatenate([zlo, zhi], axis=1)
    z_ref[...] = z

    @pl.when(pl.program_id(0) == 0)
    def _():
        cs_ref[...] = jnp.zeros_like(cs_ref)

    cs_ref[...] += jnp.sum(z, axis=0, keepdims=True)


def _finalA(acc, dv, b):
    return pl.pallas_call(
        _finalA_body,
        grid=(NB,),
        in_specs=[
            pl.BlockSpec((RB, HALF), lambda i: (i, 0)),
            pl.BlockSpec((RB, HALF), lambda i: (NB + i, 0)),
            pl.BlockSpec((RB, 1), lambda i: (i, 0)),
            pl.BlockSpec((1, D), lambda i: (0, 0)),
        ],
        out_specs=[
            pl.BlockSpec((RB, D), lambda i: (i, 0)),
            pl.BlockSpec((1, D), lambda i: (0, 0)),
        ],
        out_shape=[
            jax.ShapeDtypeStruct((N, D), jnp.float32),
            jax.ShapeDtypeStruct((1, D), jnp.float32),
        ],
    )(acc, acc, dv, b)


def _finalB_body(alo_ref, ahi_ref, dv_ref, b_ref, z_ref):
    dv = dv_ref[...]
    zlo = dv * alo_ref[...] + b_ref[0:1, 0:HALF]
    zhi = dv * ahi_ref[...] + b_ref[0:1, HALF:D]
    z_ref[...] = jnp.concatenate([zlo, zhi], axis=1)


def _finalB(acc, dv, b):
    return pl.pallas_call(
        _finalB_body,
        grid=(NB,),
        in_specs=[
            pl.BlockSpec((RB, HALF), lambda i: (i, 0)),
            pl.BlockSpec((RB, HALF), lambda i: (NB + i, 0)),
            pl.BlockSpec((RB, 1), lambda i: (i, 0)),
            pl.BlockSpec((1, D), lambda i: (0, 0)),
        ],
        out_specs=pl.BlockSpec((RB, D), lambda i: (i, 0)),
        out_shape=jax.ShapeDtypeStruct((N, D), jnp.float32),
    )(acc, acc, dv, b)


def _score1_body(z_ref, cs_ref, p_ref):
    sm = cs_ref[...] * (1.0 / N)
    dn = (((1,), (1,)), ((), ()))
    p = lax.dot_general(z_ref[...], sm, dn, preferred_element_type=jnp.float32)
    p_ref[...] = 1.0 / (1.0 + jnp.exp(-p))


def _score1(z, colsum):
    return pl.pallas_call(
        _score1_body,
        grid=(NB,),
        in_specs=[
            pl.BlockSpec((RB, D), lambda i: (i, 0)),
            pl.BlockSpec((1, D), lambda i: (0, 0)),
        ],
        out_specs=pl.BlockSpec((RB, 1), lambda i: (i, 0)),
        out_shape=jax.ShapeDtypeStruct((N, 1), jnp.float32),
    )(z, colsum)


def kernel(x, edge_index, batch, x_corrupted, edge_index_corrupted,
           batch_corrupted, W1, b1, W2, b2):
    src, dst = edge_index[0], edge_index[1]
    src_c, dst_c = edge_index_corrupted[0], edge_index_corrupted[1]

    dst2 = jnp.concatenate([dst, dst_c + ND])
    degp = _deg_kernel(dst2)
    dvA = lax.rsqrt(degp[0:N] + degp[2 * ND:2 * ND + N] + 1.0)[:, None]
    dvB = lax.rsqrt(degp[ND:ND + N] + degp[3 * ND:3 * ND + N] + 1.0)[:, None]

    b1r = b1.reshape(1, D)
    b2r = b2.reshape(1, D)

    srcxA = jnp.concatenate([src, src + N])
    srcxB = jnp.concatenate([src_c, src_c + N])

    gA1 = _mm_scale(x, W1, dvA)
    accA1 = _scatter_kernel(gA1, srcxA, dst)
    gB1 = _mm_scale(x_corrupted, W1, dvB)
    accB1 = _scatter_kernel(gB1, srcxB, dst_c)
    gA2 = _combine_mm(accA1, dvA, b1r, W2)
    accA2 = _scatter_kernel(gA2, srcxA, dst)
    gB2 = _combine_mm(accB1, dvB, b1r, W2)
    accB2 = _scatter_kernel(gB2, srcxB, dst_c)
    z, colsum = _finalA(accA2, dvA, b2r)
    pos = _score1(z, colsum)
    z_c = _finalB(accB2, dvB, b2r)
    neg = _score1(z_c, colsum)
    return pos[:, 0], neg[:, 0], z


# speedup vs baseline: 1.1344x; 1.0368x over previous
import functools

import jax
import jax.numpy as jnp
from jax import lax
from jax.experimental import pallas as pl
from jax.experimental.pallas import tpu as pltpu
from jax.experimental.pallas import tpu_sc as plsc

N = 10000
D = 256
E = 160000
HALF = 128
NC = 2
NS = 16
NW = NC * NS

ND = 10240
NDS = ND // NS
EPW = E // NW
CH = 128
DFULL = EPW // CH
DTAIL = EPW - DFULL * CH

EPS = E // NS
NFULL = EPS // CH
TAIL = EPS - NFULL * CH
RPT = 624
RREM = N - NS * RPT

RB = 1000
NB = N // RB

@functools.cache
def _mesh():
    return plsc.VectorSubcoreMesh(core_axis_name="c", subcore_axis_name="s",
                                  num_cores=NC, num_subcores=NS)


_DCH = 2 * DFULL
_DSL = 2 * ND // NS


def _deg_body(dst2, out, deg_sh, idx0, idx1, idxtA, idxtB, ones_v, zero_v,
              semi0, semi1):
    c = lax.axis_index("c")
    s = lax.axis_index("s")
    w = s * NC + c
    wE = w * EPW

    def fill_ones(i, _):
        ones_v[pl.ds(i * 16, 16)] = jnp.full((16,), 1.0, jnp.float32)
        return 0

    lax.fori_loop(0, CH // 16, fill_ones, 0)

    def fill_zero(i, _):
        zero_v[pl.ds(i * 16, 16)] = jnp.zeros((16,), jnp.float32)
        return 0

    lax.fori_loop(0, _DSL // 16, fill_zero, 0)

    def cbase(t):
        return jnp.where(t < DFULL, wE + t * CH, E + wE + (t - DFULL) * CH)

    def start_idx(t, buf, sem):
        pltpu.async_copy(dst2.at[pl.ds(cbase(t), CH)], buf, sem)

    def wait_idx(t, buf, sem):
        pltpu.make_async_copy(dst2.at[pl.ds(cbase(t), CH)], buf, sem).wait()

    start_idx(0, idx0, semi0)
    pltpu.sync_copy(zero_v, deg_sh.at[pl.ds(s * _DSL, _DSL)])
    plsc.subcore_barrier()

    def body(u, _):
        t0 = 2 * u
        start_idx(t0 + 1, idx1, semi1)
        wait_idx(t0, idx0, semi0)
        pltpu.sync_copy(ones_v, deg_sh.at[idx0], add=True)

        @pl.when(u < _DCH // 2 - 1)
        def _():
            start_idx(t0 + 2, idx0, semi0)

        wait_idx(t0 + 1, idx1, semi1)
        pltpu.sync_copy(ones_v, deg_sh.at[idx1], add=True)
        return 0

    lax.fori_loop(0, _DCH // 2, body, 0)

    pltpu.sync_copy(dst2.at[pl.ds(wE + DFULL * CH, DTAIL)], idxtA)
    pltpu.sync_copy(dst2.at[pl.ds(E + wE + DFULL * CH, DTAIL)], idxtB)
    pltpu.sync_copy(ones_v.at[pl.ds(0, DTAIL)], deg_sh.at[idxtA], add=True)
    pltpu.sync_copy(ones_v.at[pl.ds(0, DTAIL)], deg_sh.at[idxtB], add=True)

    plsc.subcore_barrier()
    pltpu.sync_copy(deg_sh.at[pl.ds(s * _DSL, _DSL)],
                    out.at[pl.ds(c * 2 * ND + s * _DSL, _DSL)])


_NBUF = 3
_URND = NFULL // _NBUF


def _scatter_body(g2, srcx, dst, out, acc_sh, sidx, didx, rows, sidxt,
                  didxt, semg, sems):
    c = lax.axis_index("c")
    s = lax.axis_index("s")
    r0 = s * RPT
    base0 = s * EPS
    cE = c * E

    def load_idx(b, j):
        pltpu.sync_copy(srcx.at[pl.ds(cE + b, CH)], sidx[j])
        pltpu.sync_copy(dst.at[pl.ds(b, CH)], didx[j])

    def start_gather(j):
        pltpu.async_copy(g2.at[sidx[j]], rows[j], semg[j])

    def wait_gather(j):
        pltpu.make_async_copy(g2.at[sidx[j]], rows[j], semg[j]).wait()

    def start_scatter(j):
        pltpu.async_copy(rows[j], acc_sh.at[didx[j]], sems[j], add=True)

    def wait_scatter(j):
        pltpu.make_async_copy(rows[j], acc_sh.at[didx[j]], sems[j]).wait()

    for j in range(_NBUF - 1):
        load_idx(base0 + j * CH, j)
        start_gather(j)

    pltpu.sync_copy(g2.at[pl.ds(c * N + r0, RPT)], acc_sh.at[pl.ds(r0, RPT)])

    @pl.when(s == NS - 1)
    def _():
        pltpu.sync_copy(g2.at[pl.ds(c * N + NS * RPT, RREM)],
                        acc_sh.at[pl.ds(NS * RPT, RREM)])

    plsc.subcore_barrier()

    def body(u, _):
        for j in range(_NBUF):
            t_is_0 = (u == 0) & (j == 0)
            last_ok = jnp.logical_or(u < _URND - 1, j == 0)
            wait_gather(j)
            start_scatter(j)
            b2 = (j + _NBUF - 1) % _NBUF

            @pl.when(jnp.logical_not(t_is_0))
            def _():
                wait_scatter(b2)

            @pl.when(last_ok)
            def _():
                load_idx(base0 + (_NBUF * u + j + _NBUF - 1) * CH, b2)
                start_gather(b2)
        return 0

    lax.fori_loop(0, _URND, body, 0)
    wait_scatter((NFULL - 1) % _NBUF)

    bt = base0 + NFULL * CH
    pltpu.sync_copy(srcx.at[pl.ds(cE + bt, TAIL)], sidxt)
    pltpu.sync_copy(dst.at[pl.ds(bt, TAIL)], didxt)
    pltpu.async_copy(g2.at[sidxt], rows[0].at[pl.ds(0, TAIL)], semg[0]).wait()
    pltpu.sync_copy(rows[0].at[pl.ds(0, TAIL)], acc_sh.at[didxt], add=True)

    plsc.subcore_barrier()
    pltpu.sync_copy(acc_sh.at[pl.ds(r0, RPT)], out.at[pl.ds(c * N + r0, RPT)])

    @pl.when(s == NS - 1)
    def _():
        pltpu.sync_copy(acc_sh.at[pl.ds(NS * RPT, RREM)],
                        out.at[pl.ds(c * N + NS * RPT, RREM)])


@functools.cache
def _deg_kernel_fn():
    return pl.kernel(
        _deg_body,
        out_type=jax.ShapeDtypeStruct((NC * 2 * ND,), jnp.float32),
        mesh=_mesh(),
        scratch_types=[
            pltpu.VMEM_SHARED((2 * ND,), jnp.float32),
            pltpu.VMEM((CH,), jnp.int32),
            pltpu.VMEM((CH,), jnp.int32),
            pltpu.VMEM((DTAIL,), jnp.int32),
            pltpu.VMEM((DTAIL,), jnp.int32),
            pltpu.VMEM((CH,), jnp.float32),
            pltpu.VMEM((_DSL,), jnp.float32),
            pltpu.SemaphoreType.DMA,
            pltpu.SemaphoreType.DMA,
        ],
    )


def _deg_kernel(dst2):
    return _deg_kernel_fn()(dst2)


def _scatter_kernel(g2, srcx, dst):
    return _scatter_kernel_fn()(g2, srcx, dst)


@functools.cache
def _scatter_kernel_fn():
    return pl.kernel(
        _scatter_body,
        out_type=jax.ShapeDtypeStruct((2 * N, HALF), jnp.float32),
        mesh=_mesh(),
        scratch_types=[
            pltpu.VMEM_SHARED((N, HALF), jnp.float32),
            [pltpu.VMEM((CH,), jnp.int32)] * _NBUF,
            [pltpu.VMEM((CH,), jnp.int32)] * _NBUF,
            [pltpu.VMEM((CH, HALF), jnp.float32)] * _NBUF,
            pltpu.VMEM((TAIL,), jnp.int32),
            pltpu.VMEM((TAIL,), jnp.int32),
            [pltpu.SemaphoreType.DMA] * _NBUF,
            [pltpu.SemaphoreType.DMA] * _NBUF,
        ],
    )


def _mm_raw_body(x_ref, w_ref, o_ref):
    o_ref[...] = jnp.dot(x_ref[...], w_ref[...],
                         preferred_element_type=jnp.float32)


def _mm_raw(xg, W):
    return pl.pallas_call(
        _mm_raw_body,
        grid=(NB,),
        in_specs=[
            pl.BlockSpec((RB, D), lambda i: (i, 0)),
            pl.BlockSpec((D, D), lambda i: (0, 0)),
        ],
        out_specs=pl.BlockSpec((RB, D), lambda i: (i, 0)),
        out_shape=jax.ShapeDtypeStruct((N, D), jnp.float32),
    )(xg, W)


def _scale_split_body(h_ref, dv_ref, o_ref):
    h = h_ref[...] * dv_ref[...]
    o_ref[0, :, :] = h[:, 0:HALF]
    o_ref[1, :, :] = h[:, HALF:D]


def _scale_split(h, dv):
    return pl.pallas_call(
        _scale_split_body,
        grid=(NB,),
        in_specs=[
            pl.BlockSpec((RB, D), lambda i: (i, 0)),
            pl.BlockSpec((RB, 1), lambda i: (i, 0)),
        ],
        out_specs=pl.BlockSpec((2, RB, HALF), lambda i: (0, i, 0)),
        out_shape=jax.ShapeDtypeStruct((2, N, HALF), jnp.float32),
    )(h, dv)


def _combine_mm_body(alo_ref, ahi_ref, dv_ref, b_ref, w_ref, o_ref):
    dv = dv_ref[...]
    acc = jnp.concatenate([alo_ref[...], ahi_ref[...]], axis=1)
    h = jnp.maximum(dv * acc + b_ref[...], 0.0)
    o = jnp.dot(h, w_ref[...], preferred_element_type=jnp.float32) * dv
    o_ref[0, :, :] = o[:, 0:HALF]
    o_ref[1, :, :] = o[:, HALF:D]


def _combine_mm(acc, dv, b, W):
    return pl.pallas_call(
        _combine_mm_body,
        grid=(NB,),
        in_specs=[
            pl.BlockSpec((RB, HALF), lambda i: (i, 0)),
            pl.BlockSpec((RB, HALF), lambda i: (NB + i, 0)),
            pl.BlockSpec((RB, 1), lambda i: (i, 0)),
            pl.BlockSpec((1, D), lambda i: (0, 0)),
            pl.BlockSpec((D, D), lambda i: (0, 0)),
        ],
        out_specs=pl.BlockSpec((2, RB, HALF), lambda i: (0, i, 0)),
        out_shape=jax.ShapeDtypeStruct((2, N, HALF), jnp.float32),
    )(acc, acc, dv, b, W)


def _finalA_body(alo_ref, ahi_ref, dv_ref, b_ref, z_ref, cs_ref):
    dv = dv_ref[...]
    acc = jnp.concatenate([alo_ref[...], ahi_ref[...]], axis=1)
    z = dv * acc + b_ref[...]
    z_ref[...] = z

    @pl.when(pl.program_id(0) == 0)
    def _():
        cs_ref[...] = jnp.zeros_like(cs_ref)

    cs_ref[...] += jnp.sum(z, axis=0, keepdims=True)


def _finalA(acc, dv, b):
    return pl.pallas_call(
        _finalA_body,
        grid=(NB,),
        in_specs=[
            pl.BlockSpec((RB, HALF), lambda i: (i, 0)),
            pl.BlockSpec((RB, HALF), lambda i: (NB + i, 0)),
            pl.BlockSpec((RB, 1), lambda i: (i, 0)),
            pl.BlockSpec((1, D), lambda i: (0, 0)),
        ],
        out_specs=[
            pl.BlockSpec((RB, D), lambda i: (i, 0)),
            pl.BlockSpec((1, D), lambda i: (0, 0)),
        ],
        out_shape=[
            jax.ShapeDtypeStruct((N, D), jnp.float32),
            jax.ShapeDtypeStruct((1, D), jnp.float32),
        ],
    )(acc, acc, dv, b)


def _finalB_body(alo_ref, ahi_ref, dv_ref, b_ref, cs_ref, z_ref, n_ref):
    dv = dv_ref[...]
    acc = jnp.concatenate([alo_ref[...], ahi_ref[...]], axis=1)
    z = dv * acc + b_ref[...]
    z_ref[...] = z
    sm = cs_ref[...] * (1.0 / N)
    dn = (((1,), (1,)), ((), ()))
    p = lax.dot_general(z, sm, dn, preferred_element_type=jnp.float32)
    n_ref[...] = 1.0 / (1.0 + jnp.exp(-p))


def _finalB(acc, dv, b, colsum):
    return pl.pallas_call(
        _finalB_body,
        grid=(NB,),
        in_specs=[
            pl.BlockSpec((RB, HALF), lambda i: (i, 0)),
            pl.BlockSpec((RB, HALF), lambda i: (NB + i, 0)),
            pl.BlockSpec((RB, 1), lambda i: (i, 0)),
            pl.BlockSpec((1, D), lambda i: (0, 0)),
            pl.BlockSpec((1, D), lambda i: (0, 0)),
        ],
        out_specs=[
            pl.BlockSpec((RB, D), lambda i: (i, 0)),
            pl.BlockSpec((RB, 1), lambda i: (i, 0)),
        ],
        out_shape=[
            jax.ShapeDtypeStruct((N, D), jnp.float32),
            jax.ShapeDtypeStruct((N, 1), jnp.float32),
        ],
    )(acc, acc, dv, b, colsum)


def _score1_body(z_ref, cs_ref, p_ref):
    sm = cs_ref[...] * (1.0 / N)
    dn = (((1,), (1,)), ((), ()))
    p = lax.dot_general(z_ref[...], sm, dn, preferred_element_type=jnp.float32)
    p_ref[...] = 1.0 / (1.0 + jnp.exp(-p))


def _score1(z, colsum):
    return pl.pallas_call(
        _score1_body,
        grid=(NB,),
        in_specs=[
            pl.BlockSpec((RB, D), lambda i: (i, 0)),
            pl.BlockSpec((1, D), lambda i: (0, 0)),
        ],
        out_specs=pl.BlockSpec((RB, 1), lambda i: (i, 0)),
        out_shape=jax.ShapeDtypeStruct((N, 1), jnp.float32),
    )(z, colsum)


def kernel(x, edge_index, batch, x_corrupted, edge_index_corrupted,
           batch_corrupted, W1, b1, W2, b2):
    src, dst = edge_index[0], edge_index[1]
    src_c, dst_c = edge_index_corrupted[0], edge_index_corrupted[1]

    dst2 = jnp.concatenate([dst, dst_c + ND])
    degp = _deg_kernel(dst2)
    dvA = lax.rsqrt(degp[0:N] + degp[2 * ND:2 * ND + N] + 1.0)[:, None]
    dvB = lax.rsqrt(degp[ND:ND + N] + degp[3 * ND:3 * ND + N] + 1.0)[:, None]

    b1r = b1.reshape(1, D)
    b2r = b2.reshape(1, D)

    srcxA = jnp.concatenate([src, src + N])
    srcxB = jnp.concatenate([src_c, src_c + N])

    hA1 = _mm_raw(x, W1)
    hB1 = _mm_raw(x_corrupted, W1)
    gA1 = _scale_split(hA1, dvA).reshape(2 * N, HALF)
    accA1 = _scatter_kernel(gA1, srcxA, dst)
    gB1 = _scale_split(hB1, dvB).reshape(2 * N, HALF)
    accB1 = _scatter_kernel(gB1, srcxB, dst_c)
    gA2 = _combine_mm(accA1, dvA, b1r, W2).reshape(2 * N, HALF)
    accA2 = _scatter_kernel(gA2, srcxA, dst)
    gB2 = _combine_mm(accB1, dvB, b1r, W2).reshape(2 * N, HALF)
    accB2 = _scatter_kernel(gB2, srcxB, dst_c)
    z, colsum = _finalA(accA2, dvA, b2r)
    pos = _score1(z, colsum)
    z_c, neg = _finalB(accB2, dvB, b2r, colsum)
    return pos[:, 0], neg[:, 0], z
